# baseline (speedup 1.0000x reference)
import functools
import math

import jax
import jax.numpy as jnp
from jax import lax
from jax.experimental import pallas as pl
from jax.experimental.pallas import tpu as pltpu
from jax.experimental.pallas import tpu_sc as plsc

_K = 16
_TD = 16
_TH = 8


def _mm(x, w, b, relu=False, br=512):
    rows, cin = x.shape
    f = w.shape[1]

    def body(x_ref, w_ref, b_ref, o_ref):
        acc = jnp.dot(x_ref[...], w_ref[...], preferred_element_type=jnp.float32)
        acc = acc + b_ref[...]
        if relu:
            acc = jnp.maximum(acc, 0.0)
        o_ref[...] = acc

    return pl.pallas_call(
        body,
        grid=(rows // br,),
        in_specs=[
            pl.BlockSpec((br, cin), lambda i: (i, 0)),
            pl.BlockSpec((cin, f), lambda i: (0, 0)),
            pl.BlockSpec((1, f), lambda i: (0, 0)),
        ],
        out_specs=pl.BlockSpec((br, f), lambda i: (i, 0)),
        out_shape=jax.ShapeDtypeStruct((rows, f), jnp.float32),
    )(x, w, b.reshape(1, f))


def _enc(t, br=2048):
    rows = t.shape[0]

    def body(t_ref, o_ref):
        col = lax.broadcasted_iota(jnp.int32, (br, _TD), 1)
        i = jnp.where(col < _TH, col, col - _TH).astype(jnp.float32)
        freq = jnp.exp(i * (-math.log(10000.0) / _TH))
        ang = t_ref[...] * freq
        o_ref[...] = jnp.where(col < _TH, jnp.sin(ang), jnp.cos(ang))

    return pl.pallas_call(
        body,
        grid=(rows // br,),
        in_specs=[pl.BlockSpec((br, 1), lambda i: (i, 0))],
        out_specs=pl.BlockSpec((br, _TD), lambda i: (i, 0)),
        out_shape=jax.ShapeDtypeStruct((rows, _TD), jnp.float32),
    )(t)


def _knn(qp, kpt, ndim, bq=512):
    b, m, _ = qp.shape
    n = kpt.shape[2]

    def body(q_ref, kt_ref, o_ref):
        q = q_ref[0]
        kt = kt_ref[0]
        if ndim == 1:
            d = jnp.abs(q[:, 0:1] - kt[0:1, :])
        else:
            d = None
            for c in range(ndim):
                diff = q[:, c : c + 1] - kt[c : c + 1, :]
                d = diff * diff if d is None else d + diff * diff
        iota = lax.broadcasted_iota(jnp.int32, (bq, n), 1)
        col16 = lax.broadcasted_iota(jnp.int32, (bq, _K), 1)
        acc = jnp.zeros((bq, _K), jnp.int32)
        big = jnp.float32(jnp.inf)
        for j in range(_K):
            mn = jnp.min(d, axis=1, keepdims=True)
            sel = jnp.min(jnp.where(d == mn, iota, n), axis=1, keepdims=True)
            acc = jnp.where(col16 == j, sel, acc)
            d = jnp.where(iota == sel, big, d)
        o_ref[0] = acc

    return pl.pallas_call(
        body,
        grid=(b, m // bq),
        in_specs=[
            pl.BlockSpec((1, bq, 8), lambda i, j: (i, j, 0)),
            pl.BlockSpec((1, 8, n), lambda i, j: (i, 0, 0)),
        ],
        out_specs=pl.BlockSpec((1, bq, _K), lambda i, j: (i, j, 0)),
        out_shape=jax.ShapeDtypeStruct((b, m, _K), jnp.int32),
    )(qp, kpt)


def _sc_gather(table, idx):
    v, d = table.shape
    r = idx.shape[0]
    nw = 32
    per_w = r // nw
    sub = 128
    nsub = per_w // sub
    grp = 1
    while grp * 2 <= nsub and (grp * 2) * sub * d <= 65536 and nsub % (grp * 2) == 0:
        grp *= 2
    ch = grp * sub
    n_ch = per_w // ch
    mesh = plsc.VectorSubcoreMesh(core_axis_name="c", subcore_axis_name="s")

    @functools.partial(
        pl.kernel,
        mesh=mesh,
        out_type=jax.ShapeDtypeStruct((r, d), jnp.float32),
        scratch_types=[
            pltpu.VMEM((per_w,), jnp.int32),
            pltpu.VMEM((ch, d), jnp.float32),
            pltpu.SemaphoreType.DMA,
        ],
    )
    def k(table_hbm, idx_hbm, out_hbm, idx_v, rows_v, sem):
        wid = lax.axis_index("s") * 2 + lax.axis_index("c")
        base = wid * per_w
        pltpu.sync_copy(idx_hbm.at[pl.ds(base, per_w)], idx_v)
        for c in range(n_ch):
            handles = []
            for g in range(grp):
                handles.append(
                    pltpu.async_copy(
                        table_hbm.at[idx_v.at[pl.ds(c * ch + g * sub, sub)]],
                        rows_v.at[pl.ds(g * sub, sub)],
                        sem,
                    )
                )
            for h in handles:
                h.wait()
            pltpu.sync_copy(rows_v, out_hbm.at[pl.ds(base + c * ch, ch)])

    return k(table, idx)


def _edge(gath, q_off, w2, b2, qc=None, qs=None, wa=None, wb=None, rq=32):
    rows = gath.shape[0]
    h = w2.shape[0]
    blk = rq * _K
    trig = qc is not None

    def body(*refs):
        if trig:
            g_ref, q_ref, w2_ref, b2_ref, qc_ref, qs_ref, wa_ref, wb_ref, o_ref = refs
        else:
            g_ref, q_ref, w2_ref, b2_ref, o_ref = refs
        g = g_ref[...]
        ri = lax.broadcasted_iota(jnp.int32, (blk, rq), 0) // _K
        ci = lax.broadcasted_iota(jnp.int32, (blk, rq), 1)
        e = (ri == ci).astype(jnp.float32)
        l1 = g[:, :h] + jnp.dot(e, q_ref[...], preferred_element_type=jnp.float32)
        if trig:
            eqc = jnp.dot(e, qc_ref[...], preferred_element_type=jnp.float32)
            eqs = jnp.dot(e, qs_ref[...], preferred_element_type=jnp.float32)
            en = g[:, h : h + _TD]
            l1 = l1 + jnp.dot(en * eqc, wa_ref[...], preferred_element_type=jnp.float32)
            l1 = l1 + jnp.dot(en * eqs, wb_ref[...], preferred_element_type=jnp.float32)
        h1 = jnp.maximum(l1, 0.0)
        h2 = jnp.dot(h1, w2_ref[...], preferred_element_type=jnp.float32) + b2_ref[...]
        h2 = jnp.maximum(h2, 0.0)
        pr = lax.broadcasted_iota(jnp.int32, (rq, blk), 0)
        pc = lax.broadcasted_iota(jnp.int32, (rq, blk), 1) // _K
        p = jnp.where(pr == pc, 1.0 / _K, 0.0)
        o_ref[...] = jnp.dot(p, h2, preferred_element_type=jnp.float32)

    in_specs = [
        pl.BlockSpec((blk, 128), lambda i: (i, 0)),
        pl.BlockSpec((rq, h), lambda i: (i, 0)),
        pl.BlockSpec((h, h), lambda i: (0, 0)),
        pl.BlockSpec((1, h), lambda i: (0, 0)),
    ]
    args = [gath, q_off, w2, b2.reshape(1, h)]
    if trig:
        in_specs += [
            pl.BlockSpec((rq, _TD), lambda i: (i, 0)),
            pl.BlockSpec((rq, _TD), lambda i: (i, 0)),
            pl.BlockSpec((_TD, h), lambda i: (0, 0)),
            pl.BlockSpec((_TD, h), lambda i: (0, 0)),
        ]
        args += [qc, qs, wa, wb]
    return pl.pallas_call(
        body,
        grid=(rows // blk,),
        in_specs=in_specs,
        out_specs=pl.BlockSpec((rq, h), lambda i: (i, 0)),
        out_shape=jax.ShapeDtypeStruct((rows // _K, h), jnp.float32),
    )(*args)


def _pad8(x):
    c = x.shape[-1]
    return jnp.concatenate(
        [x, jnp.zeros(x.shape[:-1] + (8 - c,), jnp.float32)], axis=-1
    )


def kernel(data, ids, space_pts, time_pts, query_pts, params):
    del ids
    b, n, feat = data.shape
    q = query_pts.shape[1]

    sp8 = _pad8(space_pts)
    idx_s = _knn(sp8, jnp.swapaxes(sp8, 1, 2), ndim=3)
    tp_q = _pad8(time_pts[..., None])
    tp_k = jnp.swapaxes(tp_q, 1, 2)
    idx_t = _knn(tp_q, tp_k, ndim=1)
    idx_g = _knn(_pad8(query_pts[..., None]), tp_k, ndim=1)

    offs = (jnp.arange(b, dtype=jnp.int32) * n)[:, None, None]
    fidx_s = (idx_s + offs).reshape(-1)
    fidx_t = (idx_t + offs).reshape(-1)
    fidx_g = (idx_g + offs).reshape(-1)

    enc_t = _enc(time_pts.reshape(-1, 1))
    enc_q = _enc(query_pts.reshape(-1, 1))
    qc_t = jnp.concatenate([enc_t[:, _TH:], enc_t[:, _TH:]], 1)
    qs_t = jnp.concatenate([enc_t[:, :_TH], enc_t[:, :_TH]], 1)
    qc_g = jnp.concatenate([enc_q[:, _TH:], enc_q[:, _TH:]], 1)
    qs_g = jnp.concatenate([enc_q[:, :_TH], enc_q[:, :_TH]], 1)

    kf = data.reshape(b * n, feat)
    sp = space_pts.reshape(b * n, 3)
    z32 = jnp.zeros((32,), jnp.float32)

    def pad128(*cols):
        x = jnp.concatenate(cols, 1)
        return jnp.concatenate(
            [x, jnp.zeros((x.shape[0], 128 - x.shape[1]), jnp.float32)], 1
        )

    for lvl in range(len(params["comb"])):
        cin = kf.shape[1]
        (w1s, b1s), (w2s, b2s), (w3s, b3s) = params["space"][lvl]
        t_s = _mm(jnp.concatenate([sp, kf, sp], 1), w1s, z32)
        o_s = _mm(sp, -w1s[cin + 3 :], b1s)
        g_s = _sc_gather(pad128(t_s), fidx_s)
        m_s = _edge(g_s, o_s, w2s, b2s)
        s_nei = _mm(m_s, w3s, b3s)

        (w1t, b1t), (w2t, b2t), (w3t, b3t) = params["time"][lvl]
        ct = cin + 64 + _TD
        t_t = _mm(jnp.concatenate([enc_t, kf, s_nei], 1), w1t[:ct], z32)
        w1b = w1t[ct:]
        wb = jnp.concatenate([w1b[_TH:], -w1b[:_TH]], 0)
        g1_t = _sc_gather(pad128(t_t, enc_t), fidx_t)
        o_t = jnp.broadcast_to(b1t[None, :], (b * n, 32))
        m_t = _edge(g1_t, o_t, w2t, b2t, qc=qc_t, qs=qs_t, wa=w1b, wb=wb)
        t_nei = _mm(m_t, w3t, b3t)

        (wc1, bc1), (wc2, bc2) = params["comb"][lvl]
        comb = jnp.concatenate([kf, s_nei, t_nei], 1)
        kf = _mm(_mm(comb, wc1, bc1, relu=True), wc2, bc2)

    (w1g, b1g), (w2g, b2g), (w3g, b3g) = params["target"]
    cg = _TD + kf.shape[1]
    t_g = _mm(jnp.concatenate([enc_t, kf], 1), w1g[:cg], jnp.zeros((64,), jnp.float32))
    w1bg = w1g[cg:]
    wbg = jnp.concatenate([w1bg[_TH:], -w1bg[:_TH]], 0)
    g1_g = _sc_gather(pad128(t_g, enc_t), fidx_g)
    o_g = jnp.broadcast_to(b1g[None, :], (b * q, 64))
    m_g = _edge(g1_g, o_g, w2g, b2g, qc=qc_g, qs=qs_g, wa=w1bg, wb=wbg)
    out = _mm(m_g, w3g, b3g)
    return out.reshape(b, q, -1)


# speedup vs baseline: 1.2391x; 1.2391x over previous
import functools
import math

import jax
import jax.numpy as jnp
from jax import lax
from jax.experimental import pallas as pl
from jax.experimental.pallas import tpu as pltpu
from jax.experimental.pallas import tpu_sc as plsc

_K = 16
_TD = 16
_TH = 8


def _mm(x, w, b, relu=False, br=512):
    rows, cin = x.shape
    f = w.shape[1]

    def body(x_ref, w_ref, b_ref, o_ref):
        acc = jnp.dot(x_ref[...], w_ref[...], preferred_element_type=jnp.float32)
        acc = acc + b_ref[...]
        if relu:
            acc = jnp.maximum(acc, 0.0)
        o_ref[...] = acc

    return pl.pallas_call(
        body,
        grid=(rows // br,),
        in_specs=[
            pl.BlockSpec((br, cin), lambda i: (i, 0)),
            pl.BlockSpec((cin, f), lambda i: (0, 0)),
            pl.BlockSpec((1, f), lambda i: (0, 0)),
        ],
        out_specs=pl.BlockSpec((br, f), lambda i: (i, 0)),
        out_shape=jax.ShapeDtypeStruct((rows, f), jnp.float32),
    )(x, w, b.reshape(1, f))


def _enc(t, br=2048):
    rows = t.shape[0]

    def body(t_ref, o_ref):
        col = lax.broadcasted_iota(jnp.int32, (br, _TD), 1)
        i = jnp.where(col < _TH, col, col - _TH).astype(jnp.float32)
        freq = jnp.exp(i * (-math.log(10000.0) / _TH))
        ang = t_ref[...] * freq
        o_ref[...] = jnp.where(col < _TH, jnp.sin(ang), jnp.cos(ang))

    return pl.pallas_call(
        body,
        grid=(rows // br,),
        in_specs=[pl.BlockSpec((br, 1), lambda i: (i, 0))],
        out_specs=pl.BlockSpec((br, _TD), lambda i: (i, 0)),
        out_shape=jax.ShapeDtypeStruct((rows, _TD), jnp.float32),
    )(t)


def _knn(qp, kpt, ndim, bq=512):
    b, m, _ = qp.shape
    n = kpt.shape[2]

    def body(q_ref, kt_ref, o_ref):
        q = q_ref[0]
        kt = kt_ref[0]
        if ndim == 1:
            d = jnp.abs(q[:, 0:1] - kt[0:1, :])
        else:
            d = None
            for c in range(ndim):
                diff = q[:, c : c + 1] - kt[c : c + 1, :]
                d = diff * diff if d is None else d + diff * diff
        iota = lax.broadcasted_iota(jnp.int32, (bq, n), 1)
        col16 = lax.broadcasted_iota(jnp.int32, (bq, _K), 1)
        acc = jnp.zeros((bq, _K), jnp.int32)
        big = jnp.float32(jnp.inf)
        for j in range(_K):
            mn = jnp.min(d, axis=1, keepdims=True)
            sel = jnp.min(jnp.where(d == mn, iota, n), axis=1, keepdims=True)
            acc = jnp.where(col16 == j, sel, acc)
            d = jnp.where(iota == sel, big, d)
        o_ref[0] = acc

    return pl.pallas_call(
        body,
        grid=(b, m // bq),
        in_specs=[
            pl.BlockSpec((1, bq, 8), lambda i, j: (i, j, 0)),
            pl.BlockSpec((1, 8, n), lambda i, j: (i, 0, 0)),
        ],
        out_specs=pl.BlockSpec((1, bq, _K), lambda i, j: (i, j, 0)),
        out_shape=jax.ShapeDtypeStruct((b, m, _K), jnp.int32),
    )(qp, kpt)


def _sc_gather(table, idx):
    v, d = table.shape
    r = idx.shape[0]
    nw = 32
    per_w = r // nw
    sub = 128
    nsub = per_w // sub
    grp = 1
    while grp * 2 <= nsub and (grp * 2) * sub * d <= 65536 and nsub % (grp * 2) == 0:
        grp *= 2
    ch = grp * sub
    n_ch = per_w // ch
    mesh = plsc.VectorSubcoreMesh(core_axis_name="c", subcore_axis_name="s")

    @functools.partial(
        pl.kernel,
        mesh=mesh,
        out_type=jax.ShapeDtypeStruct((r, d), jnp.float32),
        scratch_types=[
            pltpu.VMEM((per_w,), jnp.int32),
            pltpu.VMEM((ch, d), jnp.float32),
            pltpu.SemaphoreType.DMA,
        ],
    )
    def k(table_hbm, idx_hbm, out_hbm, idx_v, rows_v, sem):
        wid = lax.axis_index("s") * 2 + lax.axis_index("c")
        base = wid * per_w
        pltpu.sync_copy(idx_hbm.at[pl.ds(base, per_w)], idx_v)
        for c in range(n_ch):
            handles = []
            for g in range(grp):
                handles.append(
                    pltpu.async_copy(
                        table_hbm.at[idx_v.at[pl.ds(c * ch + g * sub, sub)]],
                        rows_v.at[pl.ds(g * sub, sub)],
                        sem,
                    )
                )
            for h in handles:
                h.wait()
            pltpu.sync_copy(rows_v, out_hbm.at[pl.ds(base + c * ch, ch)])

    return k(table, idx)


def _edge(gath, q_off, w2, b2, qc=None, qs=None, wa=None, wb=None, rq=128):
    rows = gath.shape[0]
    h = w2.shape[0]
    blk = rq * _K
    trig = qc is not None

    def body(*refs):
        if trig:
            g_ref, q_ref, w2_ref, b2_ref, qc_ref, qs_ref, wa_ref, wb_ref, o_ref = refs
        else:
            g_ref, q_ref, w2_ref, b2_ref, o_ref = refs
        g = g_ref[...]
        ri = lax.broadcasted_iota(jnp.int32, (blk, rq), 0) // _K
        ci = lax.broadcasted_iota(jnp.int32, (blk, rq), 1)
        e = (ri == ci).astype(jnp.float32)
        l1 = g[:, :h] + jnp.dot(e, q_ref[...], preferred_element_type=jnp.float32)
        if trig:
            eqc = jnp.dot(e, qc_ref[...], preferred_element_type=jnp.float32)
            eqs = jnp.dot(e, qs_ref[...], preferred_element_type=jnp.float32)
            en = g[:, h : h + _TD]
            l1 = l1 + jnp.dot(en * eqc, wa_ref[...], preferred_element_type=jnp.float32)
            l1 = l1 + jnp.dot(en * eqs, wb_ref[...], preferred_element_type=jnp.float32)
        h1 = jnp.maximum(l1, 0.0)
        h2 = jnp.dot(h1, w2_ref[...], preferred_element_type=jnp.float32) + b2_ref[...]
        h2 = jnp.maximum(h2, 0.0)
        pr = lax.broadcasted_iota(jnp.int32, (rq, blk), 0)
        pc = lax.broadcasted_iota(jnp.int32, (rq, blk), 1) // _K
        p = jnp.where(pr == pc, 1.0 / _K, 0.0)
        o_ref[...] = jnp.dot(p, h2, preferred_element_type=jnp.float32)

    in_specs = [
        pl.BlockSpec((blk, 128), lambda i: (i, 0)),
        pl.BlockSpec((rq, h), lambda i: (i, 0)),
        pl.BlockSpec((h, h), lambda i: (0, 0)),
        pl.BlockSpec((1, h), lambda i: (0, 0)),
    ]
    args = [gath, q_off, w2, b2.reshape(1, h)]
    if trig:
        in_specs += [
            pl.BlockSpec((rq, _TD), lambda i: (i, 0)),
            pl.BlockSpec((rq, _TD), lambda i: (i, 0)),
            pl.BlockSpec((_TD, h), lambda i: (0, 0)),
            pl.BlockSpec((_TD, h), lambda i: (0, 0)),
        ]
        args += [qc, qs, wa, wb]
    return pl.pallas_call(
        body,
        grid=(rows // blk,),
        in_specs=in_specs,
        out_specs=pl.BlockSpec((rq, h), lambda i: (i, 0)),
        out_shape=jax.ShapeDtypeStruct((rows // _K, h), jnp.float32),
    )(*args)


def _pad8(x):
    c = x.shape[-1]
    return jnp.concatenate(
        [x, jnp.zeros(x.shape[:-1] + (8 - c,), jnp.float32)], axis=-1
    )


def kernel(data, ids, space_pts, time_pts, query_pts, params):
    del ids
    b, n, feat = data.shape
    q = query_pts.shape[1]

    sp8 = _pad8(space_pts)
    idx_s = _knn(sp8, jnp.swapaxes(sp8, 1, 2), ndim=3)
    tp_q = _pad8(time_pts[..., None])
    tp_k = jnp.swapaxes(tp_q, 1, 2)
    idx_t = _knn(tp_q, tp_k, ndim=1)
    idx_g = _knn(_pad8(query_pts[..., None]), tp_k, ndim=1)

    offs = (jnp.arange(b, dtype=jnp.int32) * n)[:, None, None]
    fidx_s = (idx_s + offs).reshape(-1)
    fidx_t = (idx_t + offs).reshape(-1)
    fidx_g = (idx_g + offs).reshape(-1)

    enc_t = _enc(time_pts.reshape(-1, 1))
    enc_q = _enc(query_pts.reshape(-1, 1))
    qc_t = jnp.concatenate([enc_t[:, _TH:], enc_t[:, _TH:]], 1)
    qs_t = jnp.concatenate([enc_t[:, :_TH], enc_t[:, :_TH]], 1)
    qc_g = jnp.concatenate([enc_q[:, _TH:], enc_q[:, _TH:]], 1)
    qs_g = jnp.concatenate([enc_q[:, :_TH], enc_q[:, :_TH]], 1)

    kf = data.reshape(b * n, feat)
    sp = space_pts.reshape(b * n, 3)
    z32 = jnp.zeros((32,), jnp.float32)

    def pad128(*cols):
        x = jnp.concatenate(cols, 1)
        return jnp.concatenate(
            [x, jnp.zeros((x.shape[0], 128 - x.shape[1]), jnp.float32)], 1
        )

    for lvl in range(len(params["comb"])):
        cin = kf.shape[1]
        (w1s, b1s), (w2s, b2s), (w3s, b3s) = params["space"][lvl]
        t_s = _mm(jnp.concatenate([sp, kf, sp], 1), w1s, z32)
        o_s = _mm(sp, -w1s[cin + 3 :], b1s)
        g_s = _sc_gather(pad128(t_s), fidx_s)
        m_s = _edge(g_s, o_s, w2s, b2s)
        s_nei = _mm(m_s, w3s, b3s)

        (w1t, b1t), (w2t, b2t), (w3t, b3t) = params["time"][lvl]
        ct = cin + 64 + _TD
        t_t = _mm(jnp.concatenate([enc_t, kf, s_nei], 1), w1t[:ct], z32)
        w1b = w1t[ct:]
        wb = jnp.concatenate([w1b[_TH:], -w1b[:_TH]], 0)
        g1_t = _sc_gather(pad128(t_t, enc_t), fidx_t)
        o_t = jnp.broadcast_to(b1t[None, :], (b * n, 32))
        m_t = _edge(g1_t, o_t, w2t, b2t, qc=qc_t, qs=qs_t, wa=w1b, wb=wb)
        t_nei = _mm(m_t, w3t, b3t)

        (wc1, bc1), (wc2, bc2) = params["comb"][lvl]
        comb = jnp.concatenate([kf, s_nei, t_nei], 1)
        kf = _mm(_mm(comb, wc1, bc1, relu=True), wc2, bc2)

    (w1g, b1g), (w2g, b2g), (w3g, b3g) = params["target"]
    cg = _TD + kf.shape[1]
    t_g = _mm(jnp.concatenate([enc_t, kf], 1), w1g[:cg], jnp.zeros((64,), jnp.float32))
    w1bg = w1g[cg:]
    wbg = jnp.concatenate([w1bg[_TH:], -w1bg[:_TH]], 0)
    g1_g = _sc_gather(pad128(t_g, enc_t), fidx_g)
    o_g = jnp.broadcast_to(b1g[None, :], (b * q, 64))
    m_g = _edge(g1_g, o_g, w2g, b2g, qc=qc_g, qs=qs_g, wa=w1bg, wb=wbg)
    out = _mm(m_g, w3g, b3g)
    return out.reshape(b, q, -1)


# speedup vs baseline: 1.3133x; 1.0599x over previous
import functools
import math

import jax
import jax.numpy as jnp
from jax import lax
from jax.experimental import pallas as pl
from jax.experimental.pallas import tpu as pltpu
from jax.experimental.pallas import tpu_sc as plsc

_K = 16
_TD = 16
_TH = 8


def _mm(x, w, b, relu=False, br=512):
    rows, cin = x.shape
    f = w.shape[1]

    def body(x_ref, w_ref, b_ref, o_ref):
        acc = jnp.dot(x_ref[...], w_ref[...], preferred_element_type=jnp.float32)
        acc = acc + b_ref[...]
        if relu:
            acc = jnp.maximum(acc, 0.0)
        o_ref[...] = acc

    return pl.pallas_call(
        body,
        grid=(rows // br,),
        in_specs=[
            pl.BlockSpec((br, cin), lambda i: (i, 0)),
            pl.BlockSpec((cin, f), lambda i: (0, 0)),
            pl.BlockSpec((1, f), lambda i: (0, 0)),
        ],
        out_specs=pl.BlockSpec((br, f), lambda i: (i, 0)),
        out_shape=jax.ShapeDtypeStruct((rows, f), jnp.float32),
    )(x, w, b.reshape(1, f))


def _enc(t, br=2048):
    rows = t.shape[0]

    def body(t_ref, o_ref):
        col = lax.broadcasted_iota(jnp.int32, (br, _TD), 1)
        i = jnp.where(col < _TH, col, col - _TH).astype(jnp.float32)
        freq = jnp.exp(i * (-math.log(10000.0) / _TH))
        ang = t_ref[...] * freq
        o_ref[...] = jnp.where(col < _TH, jnp.sin(ang), jnp.cos(ang))

    return pl.pallas_call(
        body,
        grid=(rows // br,),
        in_specs=[pl.BlockSpec((br, 1), lambda i: (i, 0))],
        out_specs=pl.BlockSpec((br, _TD), lambda i: (i, 0)),
        out_shape=jax.ShapeDtypeStruct((rows, _TD), jnp.float32),
    )(t)


def _knn(qp, kpt, ndim, bq=512):
    b, m, _ = qp.shape
    n = kpt.shape[2]

    def body(q_ref, kt_ref, o_ref):
        q = q_ref[0]
        kt = kt_ref[0]
        if ndim == 1:
            d = jnp.abs(q[:, 0:1] - kt[0:1, :])
        else:
            d = None
            for c in range(ndim):
                diff = q[:, c : c + 1] - kt[c : c + 1, :]
                d = diff * diff if d is None else d + diff * diff
        iota = lax.broadcasted_iota(jnp.int32, (bq, n), 1)
        col16 = lax.broadcasted_iota(jnp.int32, (bq, _K), 1)
        acc = jnp.zeros((bq, _K), jnp.int32)
        big = jnp.float32(jnp.inf)
        for j in range(_K):
            mn = jnp.min(d, axis=1, keepdims=True)
            sel = jnp.min(jnp.where(d == mn, iota, n), axis=1, keepdims=True)
            acc = jnp.where(col16 == j, sel, acc)
            d = jnp.where(iota == sel, big, d)
        o_ref[0] = acc

    return pl.pallas_call(
        body,
        grid=(b, m // bq),
        in_specs=[
            pl.BlockSpec((1, bq, 8), lambda i, j: (i, j, 0)),
            pl.BlockSpec((1, 8, n), lambda i, j: (i, 0, 0)),
        ],
        out_specs=pl.BlockSpec((1, bq, _K), lambda i, j: (i, j, 0)),
        out_shape=jax.ShapeDtypeStruct((b, m, _K), jnp.int32),
    )(qp, kpt)


def _sc_gather(table, idx):
    v, d = table.shape
    r = idx.shape[0]
    nw = 32
    per_w = r // nw
    sub = 128
    nsub = per_w // sub
    grp = 1
    while grp * 2 <= nsub and (grp * 2) * sub * d <= 65536 and nsub % (grp * 2) == 0:
        grp *= 2
    ch = grp * sub
    n_ch = per_w // ch
    mesh = plsc.VectorSubcoreMesh(core_axis_name="c", subcore_axis_name="s")

    @functools.partial(
        pl.kernel,
        mesh=mesh,
        out_type=jax.ShapeDtypeStruct((r, d), jnp.float32),
        scratch_types=[
            pltpu.VMEM((per_w,), jnp.int32),
            pltpu.VMEM((ch, d), jnp.float32),
            pltpu.SemaphoreType.DMA,
        ],
    )
    def k(table_hbm, idx_hbm, out_hbm, idx_v, rows_v, sem):
        wid = lax.axis_index("s") * 2 + lax.axis_index("c")
        base = wid * per_w
        pltpu.sync_copy(idx_hbm.at[pl.ds(base, per_w)], idx_v)
        for c in range(n_ch):
            handles = []
            for g in range(grp):
                handles.append(
                    pltpu.async_copy(
                        table_hbm.at[idx_v.at[pl.ds(c * ch + g * sub, sub)]],
                        rows_v.at[pl.ds(g * sub, sub)],
                        sem,
                    )
                )
            for h in handles:
                h.wait()
            pltpu.sync_copy(rows_v, out_hbm.at[pl.ds(base + c * ch, ch)])

    return k(table, idx)


def _edge(gath, q_off, w2, b2, qc=None, qs=None, wa=None, wb=None, rq=128):
    rows = gath.shape[0]
    h = w2.shape[0]
    blk = rq * _K
    trig = qc is not None

    def body(*refs):
        if trig:
            g_ref, q_ref, w2_ref, b2_ref, qc_ref, qs_ref, wa_ref, wb_ref, o_ref = refs
        else:
            g_ref, q_ref, w2_ref, b2_ref, o_ref = refs
        g = g_ref[...]

        def expand(x):
            w = x.shape[1]
            return jnp.broadcast_to(x[:, None, :], (rq, _K, w)).reshape(blk, w)

        l1 = g[:, :h] + expand(q_ref[...])
        if trig:
            en = g[:, h : h + _TD]
            l1 = l1 + jnp.dot(en * expand(qc_ref[...]), wa_ref[...], preferred_element_type=jnp.float32)
            l1 = l1 + jnp.dot(en * expand(qs_ref[...]), wb_ref[...], preferred_element_type=jnp.float32)
        h1 = jnp.maximum(l1, 0.0)
        h2 = jnp.dot(h1, w2_ref[...], preferred_element_type=jnp.float32) + b2_ref[...]
        h2 = jnp.maximum(h2, 0.0)
        o_ref[...] = jnp.mean(h2.reshape(rq, _K, h), axis=1)

    in_specs = [
        pl.BlockSpec((blk, 128), lambda i: (i, 0)),
        pl.BlockSpec((rq, h), lambda i: (i, 0)),
        pl.BlockSpec((h, h), lambda i: (0, 0)),
        pl.BlockSpec((1, h), lambda i: (0, 0)),
    ]
    args = [gath, q_off, w2, b2.reshape(1, h)]
    if trig:
        in_specs += [
            pl.BlockSpec((rq, _TD), lambda i: (i, 0)),
            pl.BlockSpec((rq, _TD), lambda i: (i, 0)),
            pl.BlockSpec((_TD, h), lambda i: (0, 0)),
            pl.BlockSpec((_TD, h), lambda i: (0, 0)),
        ]
        args += [qc, qs, wa, wb]
    return pl.pallas_call(
        body,
        grid=(rows // blk,),
        in_specs=in_specs,
        out_specs=pl.BlockSpec((rq, h), lambda i: (i, 0)),
        out_shape=jax.ShapeDtypeStruct((rows // _K, h), jnp.float32),
    )(*args)


def _pad8(x):
    c = x.shape[-1]
    return jnp.concatenate(
        [x, jnp.zeros(x.shape[:-1] + (8 - c,), jnp.float32)], axis=-1
    )


def kernel(data, ids, space_pts, time_pts, query_pts, params):
    del ids
    b, n, feat = data.shape
    q = query_pts.shape[1]

    sp8 = _pad8(space_pts)
    idx_s = _knn(sp8, jnp.swapaxes(sp8, 1, 2), ndim=3)
    tp_q = _pad8(time_pts[..., None])
    tp_k = jnp.swapaxes(tp_q, 1, 2)
    idx_t = _knn(tp_q, tp_k, ndim=1)
    idx_g = _knn(_pad8(query_pts[..., None]), tp_k, ndim=1)

    offs = (jnp.arange(b, dtype=jnp.int32) * n)[:, None, None]
    fidx_s = (idx_s + offs).reshape(-1)
    fidx_t = (idx_t + offs).reshape(-1)
    fidx_g = (idx_g + offs).reshape(-1)

    enc_t = _enc(time_pts.reshape(-1, 1))
    enc_q = _enc(query_pts.reshape(-1, 1))
    qc_t = jnp.concatenate([enc_t[:, _TH:], enc_t[:, _TH:]], 1)
    qs_t = jnp.concatenate([enc_t[:, :_TH], enc_t[:, :_TH]], 1)
    qc_g = jnp.concatenate([enc_q[:, _TH:], enc_q[:, _TH:]], 1)
    qs_g = jnp.concatenate([enc_q[:, :_TH], enc_q[:, :_TH]], 1)

    kf = data.reshape(b * n, feat)
    sp = space_pts.reshape(b * n, 3)
    z32 = jnp.zeros((32,), jnp.float32)

    def pad128(*cols):
        x = jnp.concatenate(cols, 1)
        return jnp.concatenate(
            [x, jnp.zeros((x.shape[0], 128 - x.shape[1]), jnp.float32)], 1
        )

    for lvl in range(len(params["comb"])):
        cin = kf.shape[1]
        (w1s, b1s), (w2s, b2s), (w3s, b3s) = params["space"][lvl]
        t_s = _mm(jnp.concatenate([sp, kf, sp], 1), w1s, z32)
        o_s = _mm(sp, -w1s[cin + 3 :], b1s)
        g_s = _sc_gather(pad128(t_s), fidx_s)
        m_s = _edge(g_s, o_s, w2s, b2s)
        s_nei = _mm(m_s, w3s, b3s)

        (w1t, b1t), (w2t, b2t), (w3t, b3t) = params["time"][lvl]
        ct = cin + 64 + _TD
        t_t = _mm(jnp.concatenate([enc_t, kf, s_nei], 1), w1t[:ct], z32)
        w1b = w1t[ct:]
        wb = jnp.concatenate([w1b[_TH:], -w1b[:_TH]], 0)
        g1_t = _sc_gather(pad128(t_t, enc_t), fidx_t)
        o_t = jnp.broadcast_to(b1t[None, :], (b * n, 32))
        m_t = _edge(g1_t, o_t, w2t, b2t, qc=qc_t, qs=qs_t, wa=w1b, wb=wb)
        t_nei = _mm(m_t, w3t, b3t)

        (wc1, bc1), (wc2, bc2) = params["comb"][lvl]
        comb = jnp.concatenate([kf, s_nei, t_nei], 1)
        kf = _mm(_mm(comb, wc1, bc1, relu=True), wc2, bc2)

    (w1g, b1g), (w2g, b2g), (w3g, b3g) = params["target"]
    cg = _TD + kf.shape[1]
    t_g = _mm(jnp.concatenate([enc_t, kf], 1), w1g[:cg], jnp.zeros((64,), jnp.float32))
    w1bg = w1g[cg:]
    wbg = jnp.concatenate([w1bg[_TH:], -w1bg[:_TH]], 0)
    g1_g = _sc_gather(pad128(t_g, enc_t), fidx_g)
    o_g = jnp.broadcast_to(b1g[None, :], (b * q, 64))
    m_g = _edge(g1_g, o_g, w2g, b2g, qc=qc_g, qs=qs_g, wa=w1bg, wb=wbg)
    out = _mm(m_g, w3g, b3g)
    return out.reshape(b, q, -1)
